# baseline (speedup 1.0000x reference)
import functools

import jax
import jax.numpy as jnp
from jax import lax
from jax.experimental import pallas as pl
from jax.experimental.pallas import tpu as pltpu
from jax.experimental.pallas import tpu_sc as plsc

_S = 2
_NB = 512
_NTC = 12288

_NC = 2
_NS = 16
_LN2 = 0.6931471805599453


def _partial_sum(p, d, e, *, Q, K):
    NB = p.shape[0]
    zs = [p[:, q * K:(q + 1) * K] for q in range(Q)]
    s = zs[0]
    for q in range(1, Q):
        s = s + zs[q]
    zlast = 1.0 - s
    m = zlast
    for z in zs:
        m = jnp.maximum(m, z)
    se = jnp.exp(zlast - m)
    for z in zs:
        se = se + jnp.exp(z - m)
    lse = m + jnp.log(se)

    d = d.reshape(NB, 1)
    e = e.reshape(NB, 1)
    kio = jax.lax.broadcasted_iota(jnp.int32, (NB, K), 1)
    c = jnp.where(kio <= d, lse - zlast, 0.0)

    ze = zs[Q - 1]
    for q in range(Q - 2, -1, -1):
        ze = jnp.where(e == q + 1, zs[q], ze)
    corr = jnp.where((kio == d) & (e != 0), zlast - ze, 0.0)
    return jnp.sum(c) + jnp.sum(corr)


def _tc_body(*refs, Q, K):
    phi_refs = refs[:_S]
    d_refs = refs[_S:2 * _S]
    e_refs = refs[2 * _S:3 * _S]
    out_ref = refs[3 * _S]
    total = 0.0
    for s in range(_S):
        total += _partial_sum(phi_refs[s][...], d_refs[s][0, 0, :],
                              e_refs[s][0, 0, :], Q=Q, K=K)

    @pl.when(pl.program_id(0) == 0)
    def _init():
        out_ref[0, 0] = 0.0

    out_ref[0, 0] += total


def _tc_sum(phi2, d3, e3, n_rows, Q, K):
    nblk = n_rows // _NB
    g = nblk // _S

    def phi_map(s):
        return lambda i: (i + s * g, 0)

    def de_map(s):
        return lambda i: (i + s * g, 0, 0)

    out = pl.pallas_call(
        functools.partial(_tc_body, Q=Q, K=K),
        grid=(g,),
        in_specs=(
            [pl.BlockSpec((_NB, Q * K), phi_map(s)) for s in range(_S)]
            + [pl.BlockSpec((1, 1, _NB), de_map(s)) for s in range(_S)]
            + [pl.BlockSpec((1, 1, _NB), de_map(s)) for s in range(_S)]
        ),
        out_specs=pl.BlockSpec(memory_space=pltpu.SMEM),
        out_shape=jax.ShapeDtypeStruct((1, 1), jnp.float32),
        compiler_params=pltpu.CompilerParams(
            dimension_semantics=("arbitrary",),
        ),
    )(*([phi2] * _S + [d3] * _S + [e3] * _S))
    return out[0, 0]


def _sc_log(se):
    bits = lax.bitcast_convert_type(se, jnp.int32)
    E = jnp.right_shift(bits, 23) - 127
    mb = jnp.bitwise_or(jnp.bitwise_and(bits, 0x7FFFFF), 0x3F800000)
    mm = lax.bitcast_convert_type(mb, jnp.float32)
    u = (mm - 1.0) / (mm + 1.0)
    u2 = u * u
    lgm = u * (2.0 + u2 * (2.0 / 3.0 + u2 * (2.0 / 5.0 + u2 * (
        2.0 / 7.0 + u2 * (2.0 / 9.0 + u2 * (2.0 / 11.0))))))
    return E.astype(jnp.float32) * _LN2 + lgm


def _sc_partials(phi2, d, e, row0, rpt, Q, K):
    nw = _NC * _NS
    nch = rpt // 16
    assert nch % 2 == 0 and rpt % 16 == 0
    mesh = plsc.VectorSubcoreMesh(core_axis_name="c", subcore_axis_name="s")

    @functools.partial(
        pl.kernel,
        out_type=jax.ShapeDtypeStruct((nw, 16), jnp.float32),
        mesh=mesh,
        scratch_types=[
            pltpu.VMEM((2, 16, Q * K), jnp.float32),
            pltpu.VMEM((rpt,), jnp.int32),
            pltpu.VMEM((rpt,), jnp.int32),
            pltpu.VMEM((16,), jnp.float32),
            pltpu.SemaphoreType.DMA,
            pltpu.SemaphoreType.DMA,
        ],
        compiler_params=pltpu.CompilerParams(use_tc_tiling_on_sc=True),
    )
    def sc_kernel(phi_hbm, d_hbm, e_hbm, out_hbm, buf, dv, ev, accv, sem0, sem1):
        wid = lax.axis_index("s") * _NC + lax.axis_index("c")
        base = row0 + wid * rpt
        pltpu.sync_copy(d_hbm.at[pl.ds(base, rpt)], dv)
        pltpu.sync_copy(e_hbm.at[pl.ds(base, rpt)], ev)

        iota = lax.iota(jnp.int32, 16)

        def rows(c):
            return pl.ds(base + c * 16, 16)

        def process(b, c, acc):
            d16 = dv[pl.ds(c * 16, 16)]
            e16 = ev[pl.ds(c * 16, 16)]
            one = jnp.int32(1)
            zero = jnp.int32(0)

            def clamp01f(x):
                return jnp.minimum(jnp.maximum(x, zero), one).astype(
                    jnp.float32)

            for s in range(16):
                dj = d16[s]
                djv = jnp.full((16,), dj, jnp.int32)
                ejv = jnp.full((16,), e16[s], jnp.int32)
                mq = [clamp01f(one - abs(ejv - (q + 1))) for q in range(4)]
                pa = djv - iota + 1

                def kstep(t, acc, s=s, pa=pa):
                    k0 = t * 16
                    z0 = buf[b, s, pl.ds(k0, 16)]
                    z1 = buf[b, s, pl.ds(k0 + K, 16)]
                    z2 = buf[b, s, pl.ds(k0 + 2 * K, 16)]
                    z3 = buf[b, s, pl.ds(k0 + 3 * K, 16)]
                    z4 = 1.0 - (z0 + z1 + z2 + z3)
                    se = (jnp.exp(z0) + jnp.exp(z1) + jnp.exp(z2)
                          + jnp.exp(z3) + jnp.exp(z4))
                    pm = clamp01f(pa - k0)
                    return acc + pm * (_sc_log(se) - z4)

                acc = lax.fori_loop(0, K // 16, kstep, acc, unroll=4)

                kc0 = (dj // 16) * 16
                z0 = buf[b, s, pl.ds(kc0, 16)]
                z1 = buf[b, s, pl.ds(kc0 + K, 16)]
                z2 = buf[b, s, pl.ds(kc0 + 2 * K, 16)]
                z3 = buf[b, s, pl.ds(kc0 + 3 * K, 16)]
                z4 = 1.0 - (z0 + z1 + z2 + z3)
                kv = kc0 + iota
                md = clamp01f(djv - kv + 1) * clamp01f(kv - djv + 1)
                ze = mq[0] * z0 + mq[1] * z1 + mq[2] * z2 + mq[3] * z3
                mev = mq[0] + mq[1] + mq[2] + mq[3]
                acc = acc + md * (mev * z4 - ze)
            return acc

        pltpu.async_copy(phi_hbm.at[rows(0)], buf.at[0], sem0)

        def pair(p, acc):
            c0 = 2 * p
            pltpu.async_copy(phi_hbm.at[rows(c0 + 1)], buf.at[1], sem1)
            pltpu.make_async_copy(phi_hbm.at[rows(c0)], buf.at[0], sem0).wait()
            acc = process(0, c0, acc)

            @pl.when(c0 + 2 < nch)
            def _():
                pltpu.async_copy(phi_hbm.at[rows(c0 + 2)], buf.at[0], sem0)

            pltpu.make_async_copy(
                phi_hbm.at[rows(c0 + 1)], buf.at[1], sem1).wait()
            acc = process(1, c0 + 1, acc)
            return acc

        acc = lax.fori_loop(0, nch // 2, pair, jnp.zeros((16,), jnp.float32))
        accv[...] = acc
        pltpu.sync_copy(accv, out_hbm.at[wid])

    return sc_kernel(phi2, d, e)


def kernel(phi, idx_durations, events):
    N, Q, K = phi.shape
    phi2 = phi.reshape(N, Q * K)
    d = idx_durations.astype(jnp.int32)
    e = events.astype(jnp.int32)

    n_sc = N - _NTC
    parts = []
    if _NTC > 0:
        nblk = N // _NB
        d3 = d.reshape(nblk, 1, _NB)
        e3 = e.reshape(nblk, 1, _NB)
        parts.append(_tc_sum(phi2, d3, e3, _NTC, Q, K))
    if n_sc > 0:
        rpt = n_sc // (_NC * _NS)
        sc = _sc_partials(phi2[_NTC:], d[_NTC:], e[_NTC:], 0, rpt, Q, K)
        parts.append(jnp.sum(sc))
    total = parts[0]
    for p in parts[1:]:
        total = total + p
    return total / N


# speedup vs baseline: 1.0831x; 1.0831x over previous
import functools

import jax
import jax.numpy as jnp
from jax import lax
from jax.experimental import pallas as pl
from jax.experimental.pallas import tpu as pltpu
from jax.experimental.pallas import tpu_sc as plsc

_S = 2
_NB = 512
_NTC = 12288

_NC = 2
_NS = 16
_LN2 = 0.6931471805599453


def _partial_sum(p, d, e, *, Q, K):
    NB = p.shape[0]
    zs = [p[:, q * K:(q + 1) * K] for q in range(Q)]
    s = zs[0]
    for q in range(1, Q):
        s = s + zs[q]
    zlast = 1.0 - s
    m = zlast
    for z in zs:
        m = jnp.maximum(m, z)
    se = jnp.exp(zlast - m)
    for z in zs:
        se = se + jnp.exp(z - m)
    lse = m + jnp.log(se)

    d = d.reshape(NB, 1)
    e = e.reshape(NB, 1)
    kio = jax.lax.broadcasted_iota(jnp.int32, (NB, K), 1)
    c = jnp.where(kio <= d, lse - zlast, 0.0)

    ze = zs[Q - 1]
    for q in range(Q - 2, -1, -1):
        ze = jnp.where(e == q + 1, zs[q], ze)
    corr = jnp.where((kio == d) & (e != 0), zlast - ze, 0.0)
    return jnp.sum(c) + jnp.sum(corr)


def _tc_body(*refs, Q, K):
    phi_refs = refs[:_S]
    d_refs = refs[_S:2 * _S]
    e_refs = refs[2 * _S:3 * _S]
    out_ref = refs[3 * _S]
    total = 0.0
    for s in range(_S):
        total += _partial_sum(phi_refs[s][...], d_refs[s][0, 0, :],
                              e_refs[s][0, 0, :], Q=Q, K=K)

    @pl.when(pl.program_id(0) == 0)
    def _init():
        out_ref[0, 0] = 0.0

    out_ref[0, 0] += total


def _tc_sum(phi2, d3, e3, n_rows, Q, K):
    nblk = n_rows // _NB
    g = nblk // _S

    def phi_map(s):
        return lambda i: (i + s * g, 0)

    def de_map(s):
        return lambda i: (i + s * g, 0, 0)

    out = pl.pallas_call(
        functools.partial(_tc_body, Q=Q, K=K),
        grid=(g,),
        in_specs=(
            [pl.BlockSpec((_NB, Q * K), phi_map(s)) for s in range(_S)]
            + [pl.BlockSpec((1, 1, _NB), de_map(s)) for s in range(_S)]
            + [pl.BlockSpec((1, 1, _NB), de_map(s)) for s in range(_S)]
        ),
        out_specs=pl.BlockSpec(memory_space=pltpu.SMEM),
        out_shape=jax.ShapeDtypeStruct((1, 1), jnp.float32),
        compiler_params=pltpu.CompilerParams(
            dimension_semantics=("arbitrary",),
        ),
    )(*([phi2] * _S + [d3] * _S + [e3] * _S))
    return out[0, 0]


def _sc_log(se):
    bits = lax.bitcast_convert_type(se, jnp.int32)
    E = jnp.right_shift(bits, 23) - 127
    mb = jnp.bitwise_or(jnp.bitwise_and(bits, 0x7FFFFF), 0x3F800000)
    mm = lax.bitcast_convert_type(mb, jnp.float32)
    u = (mm - 1.0) / (mm + 1.0)
    u2 = u * u
    lgm = u * (2.0 + u2 * (2.0 / 3.0 + u2 * (2.0 / 5.0 + u2 * (
        2.0 / 7.0 + u2 * (2.0 / 9.0 + u2 * (2.0 / 11.0))))))
    return E.astype(jnp.float32) * _LN2 + lgm


def _sc_partials(phi2, d, e, row0, rpt, Q, K):
    nw = _NC * _NS
    nch = rpt // 16
    assert nch % 2 == 0 and rpt % 16 == 0
    mesh = plsc.VectorSubcoreMesh(core_axis_name="c", subcore_axis_name="s")

    @functools.partial(
        pl.kernel,
        out_type=jax.ShapeDtypeStruct((nw, 16), jnp.float32),
        mesh=mesh,
        scratch_types=[
            pltpu.VMEM((2, 16, Q * K), jnp.float32),
            pltpu.VMEM((rpt,), jnp.int32),
            pltpu.VMEM((rpt,), jnp.int32),
            pltpu.VMEM((16,), jnp.float32),
            pltpu.SemaphoreType.DMA,
            pltpu.SemaphoreType.DMA,
        ],
        compiler_params=pltpu.CompilerParams(use_tc_tiling_on_sc=True),
    )
    def sc_kernel(phi_hbm, d_hbm, e_hbm, out_hbm, buf, dv, ev, accv, sem0, sem1):
        wid = lax.axis_index("s") * _NC + lax.axis_index("c")
        base = row0 + wid * rpt
        pltpu.sync_copy(d_hbm.at[wid], dv)
        pltpu.sync_copy(e_hbm.at[wid], ev)

        iota = lax.iota(jnp.int32, 16)

        def rows(c):
            return pl.ds(base + c * 16, 16)

        def process(b, c, acc):
            d16 = dv[pl.ds(c * 16, 16)]
            e16 = ev[pl.ds(c * 16, 16)]
            one = jnp.int32(1)
            zero = jnp.int32(0)

            def clamp01f(x):
                return jnp.minimum(jnp.maximum(x, zero), one).astype(
                    jnp.float32)

            for s in range(16):
                dj = d16[s]
                djv = jnp.full((16,), dj, jnp.int32)
                ejv = jnp.full((16,), e16[s], jnp.int32)
                mq = [clamp01f(one - abs(ejv - (q + 1))) for q in range(4)]
                pa = djv - iota + 1

                def kstep(t, acc, s=s, pa=pa):
                    k0 = t * 16
                    z0 = buf[b, s, pl.ds(k0, 16)]
                    z1 = buf[b, s, pl.ds(k0 + K, 16)]
                    z2 = buf[b, s, pl.ds(k0 + 2 * K, 16)]
                    z3 = buf[b, s, pl.ds(k0 + 3 * K, 16)]
                    z4 = 1.0 - (z0 + z1 + z2 + z3)
                    se = (jnp.exp(z0) + jnp.exp(z1) + jnp.exp(z2)
                          + jnp.exp(z3) + jnp.exp(z4))
                    pm = clamp01f(pa - k0)
                    return acc + pm * (_sc_log(se) - z4)

                acc = lax.fori_loop(0, K // 16, kstep, acc, unroll=4)

                kc0 = (dj // 16) * 16
                z0 = buf[b, s, pl.ds(kc0, 16)]
                z1 = buf[b, s, pl.ds(kc0 + K, 16)]
                z2 = buf[b, s, pl.ds(kc0 + 2 * K, 16)]
                z3 = buf[b, s, pl.ds(kc0 + 3 * K, 16)]
                z4 = 1.0 - (z0 + z1 + z2 + z3)
                kv = kc0 + iota
                md = clamp01f(djv - kv + 1) * clamp01f(kv - djv + 1)
                ze = mq[0] * z0 + mq[1] * z1 + mq[2] * z2 + mq[3] * z3
                mev = mq[0] + mq[1] + mq[2] + mq[3]
                acc = acc + md * (mev * z4 - ze)
            return acc

        pltpu.async_copy(phi_hbm.at[rows(0)], buf.at[0], sem0)

        def pair(p, acc):
            c0 = 2 * p
            pltpu.async_copy(phi_hbm.at[rows(c0 + 1)], buf.at[1], sem1)
            pltpu.make_async_copy(phi_hbm.at[rows(c0)], buf.at[0], sem0).wait()
            acc = process(0, c0, acc)

            @pl.when(c0 + 2 < nch)
            def _():
                pltpu.async_copy(phi_hbm.at[rows(c0 + 2)], buf.at[0], sem0)

            pltpu.make_async_copy(
                phi_hbm.at[rows(c0 + 1)], buf.at[1], sem1).wait()
            acc = process(1, c0 + 1, acc)
            return acc

        acc = lax.fori_loop(0, nch // 2, pair, jnp.zeros((16,), jnp.float32))
        accv[...] = acc
        pltpu.sync_copy(accv, out_hbm.at[wid])

    return sc_kernel(phi2, d, e)


def kernel(phi, idx_durations, events):
    N, Q, K = phi.shape
    phi2 = phi.reshape(N, Q * K)
    d = idx_durations.astype(jnp.int32)
    e = events.astype(jnp.int32)

    n_sc = N - _NTC
    parts = []
    if _NTC > 0:
        nblk = N // _NB
        d3 = d.reshape(nblk, 1, _NB)
        e3 = e.reshape(nblk, 1, _NB)
        parts.append(_tc_sum(phi2, d3, e3, _NTC, Q, K))
    if n_sc > 0:
        nw = _NC * _NS
        rpt = n_sc // nw
        d2 = d[_NTC:].reshape(nw, rpt)
        e2 = e[_NTC:].reshape(nw, rpt)
        sc = _sc_partials(phi2, d2, e2, _NTC, rpt, Q, K)
        parts.append(jnp.sum(sc))
    total = parts[0]
    for p in parts[1:]:
        total = total + p
    return total / N


# speedup vs baseline: 1.2919x; 1.1928x over previous
import functools

import jax
import jax.numpy as jnp
from jax import lax
from jax.experimental import pallas as pl
from jax.experimental.pallas import tpu as pltpu
from jax.experimental.pallas import tpu_sc as plsc

_S = 2
_NB = 512
_NTC = 15360

_NC = 2
_NS = 16
_LN2 = 0.6931471805599453


def _partial_sum(p, d, e, *, Q, K):
    NB = p.shape[0]
    zs = [p[:, q * K:(q + 1) * K] for q in range(Q)]
    s = zs[0]
    for q in range(1, Q):
        s = s + zs[q]
    zlast = 1.0 - s
    m = zlast
    for z in zs:
        m = jnp.maximum(m, z)
    se = jnp.exp(zlast - m)
    for z in zs:
        se = se + jnp.exp(z - m)
    lse = m + jnp.log(se)

    d = d.reshape(NB, 1)
    e = e.reshape(NB, 1)
    kio = jax.lax.broadcasted_iota(jnp.int32, (NB, K), 1)
    c = jnp.where(kio <= d, lse - zlast, 0.0)

    ze = zs[Q - 1]
    for q in range(Q - 2, -1, -1):
        ze = jnp.where(e == q + 1, zs[q], ze)
    corr = jnp.where((kio == d) & (e != 0), zlast - ze, 0.0)
    return jnp.sum(c) + jnp.sum(corr)


def _tc_body(*refs, Q, K):
    phi_refs = refs[:_S]
    d_refs = refs[_S:2 * _S]
    e_refs = refs[2 * _S:3 * _S]
    out_ref = refs[3 * _S]
    total = 0.0
    for s in range(_S):
        total += _partial_sum(phi_refs[s][...], d_refs[s][0, 0, :],
                              e_refs[s][0, 0, :], Q=Q, K=K)

    @pl.when(pl.program_id(0) == 0)
    def _init():
        out_ref[0, 0] = 0.0

    out_ref[0, 0] += total


def _tc_sum(phi2, d3, e3, n_rows, Q, K):
    nblk = n_rows // _NB
    g = nblk // _S

    def phi_map(s):
        return lambda i: (i + s * g, 0)

    def de_map(s):
        return lambda i: (i + s * g, 0, 0)

    out = pl.pallas_call(
        functools.partial(_tc_body, Q=Q, K=K),
        grid=(g,),
        in_specs=(
            [pl.BlockSpec((_NB, Q * K), phi_map(s)) for s in range(_S)]
            + [pl.BlockSpec((1, 1, _NB), de_map(s)) for s in range(_S)]
            + [pl.BlockSpec((1, 1, _NB), de_map(s)) for s in range(_S)]
        ),
        out_specs=pl.BlockSpec(memory_space=pltpu.SMEM),
        out_shape=jax.ShapeDtypeStruct((1, 1), jnp.float32),
        compiler_params=pltpu.CompilerParams(
            dimension_semantics=("arbitrary",),
        ),
    )(*([phi2] * _S + [d3] * _S + [e3] * _S))
    return out[0, 0]


def _sc_log(se):
    bits = lax.bitcast_convert_type(se, jnp.int32)
    E = jnp.right_shift(bits, 23) - 127
    mb = jnp.bitwise_or(jnp.bitwise_and(bits, 0x7FFFFF), 0x3F800000)
    mm = lax.bitcast_convert_type(mb, jnp.float32)
    u = (mm - 1.0) / (mm + 1.0)
    u2 = u * u
    lgm = u * (2.0 + u2 * (2.0 / 3.0 + u2 * (2.0 / 5.0 + u2 * (
        2.0 / 7.0 + u2 * (2.0 / 9.0 + u2 * (2.0 / 11.0))))))
    return E.astype(jnp.float32) * _LN2 + lgm


def _sc_partials(phi2, d, e, row0, rpt, Q, K):
    nw = _NC * _NS
    nch = rpt // 16
    assert nch % 2 == 0 and rpt % 16 == 0
    mesh = plsc.VectorSubcoreMesh(core_axis_name="c", subcore_axis_name="s")

    @functools.partial(
        pl.kernel,
        out_type=jax.ShapeDtypeStruct((nw, 16), jnp.float32),
        mesh=mesh,
        scratch_types=[
            pltpu.VMEM((2, 16, Q * K), jnp.float32),
            pltpu.VMEM((rpt,), jnp.int32),
            pltpu.VMEM((rpt,), jnp.int32),
            pltpu.VMEM((16,), jnp.float32),
            pltpu.SemaphoreType.DMA,
            pltpu.SemaphoreType.DMA,
        ],
        compiler_params=pltpu.CompilerParams(use_tc_tiling_on_sc=True),
    )
    def sc_kernel(phi_hbm, d_hbm, e_hbm, out_hbm, buf, dv, ev, accv, sem0, sem1):
        wid = lax.axis_index("s") * _NC + lax.axis_index("c")
        base = row0 + wid * rpt
        pltpu.sync_copy(d_hbm.at[wid], dv)
        pltpu.sync_copy(e_hbm.at[wid], ev)

        iota = lax.iota(jnp.int32, 16)

        def rows(c):
            return pl.ds(base + c * 16, 16)

        def process(b, c, acc):
            d16 = dv[pl.ds(c * 16, 16)]
            e16 = ev[pl.ds(c * 16, 16)]
            one = jnp.int32(1)
            zero = jnp.int32(0)

            def clamp01f(x):
                return jnp.minimum(jnp.maximum(x, zero), one).astype(
                    jnp.float32)

            for s in range(16):
                dj = d16[s]
                djv = jnp.full((16,), dj, jnp.int32)
                ejv = jnp.full((16,), e16[s], jnp.int32)
                mq = [clamp01f(one - abs(ejv - (q + 1))) for q in range(4)]
                pa = djv - iota + 1

                def kstep(t, acc, s=s, pa=pa):
                    k0 = t * 16
                    z0 = buf[b, s, pl.ds(k0, 16)]
                    z1 = buf[b, s, pl.ds(k0 + K, 16)]
                    z2 = buf[b, s, pl.ds(k0 + 2 * K, 16)]
                    z3 = buf[b, s, pl.ds(k0 + 3 * K, 16)]
                    z4 = 1.0 - (z0 + z1 + z2 + z3)
                    se = (jnp.exp(z0) + jnp.exp(z1) + jnp.exp(z2)
                          + jnp.exp(z3) + jnp.exp(z4))
                    pm = clamp01f(pa - k0)
                    return acc + pm * (_sc_log(se) - z4)

                acc = lax.fori_loop(0, K // 16, kstep, acc, unroll=4)

                kc0 = (dj // 16) * 16
                z0 = buf[b, s, pl.ds(kc0, 16)]
                z1 = buf[b, s, pl.ds(kc0 + K, 16)]
                z2 = buf[b, s, pl.ds(kc0 + 2 * K, 16)]
                z3 = buf[b, s, pl.ds(kc0 + 3 * K, 16)]
                z4 = 1.0 - (z0 + z1 + z2 + z3)
                kv = kc0 + iota
                md = clamp01f(djv - kv + 1) * clamp01f(kv - djv + 1)
                ze = mq[0] * z0 + mq[1] * z1 + mq[2] * z2 + mq[3] * z3
                mev = mq[0] + mq[1] + mq[2] + mq[3]
                acc = acc + md * (mev * z4 - ze)
            return acc

        pltpu.async_copy(phi_hbm.at[rows(0)], buf.at[0], sem0)

        def pair(p, acc):
            c0 = 2 * p
            pltpu.async_copy(phi_hbm.at[rows(c0 + 1)], buf.at[1], sem1)
            pltpu.make_async_copy(phi_hbm.at[rows(c0)], buf.at[0], sem0).wait()
            acc = process(0, c0, acc)

            @pl.when(c0 + 2 < nch)
            def _():
                pltpu.async_copy(phi_hbm.at[rows(c0 + 2)], buf.at[0], sem0)

            pltpu.make_async_copy(
                phi_hbm.at[rows(c0 + 1)], buf.at[1], sem1).wait()
            acc = process(1, c0 + 1, acc)
            return acc

        acc = lax.fori_loop(0, nch // 2, pair, jnp.zeros((16,), jnp.float32))
        accv[...] = acc
        pltpu.sync_copy(accv, out_hbm.at[wid])

    return sc_kernel(phi2, d, e)


def kernel(phi, idx_durations, events):
    N, Q, K = phi.shape
    phi2 = phi.reshape(N, Q * K)
    d = idx_durations.astype(jnp.int32)
    e = events.astype(jnp.int32)

    n_sc = N - _NTC
    parts = []
    if _NTC > 0:
        nblk = N // _NB
        d3 = d.reshape(nblk, 1, _NB)
        e3 = e.reshape(nblk, 1, _NB)
        parts.append(_tc_sum(phi2, d3, e3, _NTC, Q, K))
    if n_sc > 0:
        nw = _NC * _NS
        rpt = n_sc // nw
        d2 = d[_NTC:].reshape(nw, rpt)
        e2 = e[_NTC:].reshape(nw, rpt)
        sc = _sc_partials(phi2, d2, e2, _NTC, rpt, Q, K)
        parts.append(jnp.sum(sc))
    total = parts[0]
    for p in parts[1:]:
        total = total + p
    return total / N


# speedup vs baseline: 1.3188x; 1.0208x over previous
import functools

import jax
import jax.numpy as jnp
from jax.experimental import pallas as pl
from jax.experimental.pallas import tpu as pltpu

_S = 2


def _partial_sum(p, d, e, *, Q, K):
    NB = p.shape[0]
    zs = [p[:, q * K:(q + 1) * K] for q in range(Q)]
    s = zs[0]
    for q in range(1, Q):
        s = s + zs[q]
    zlast = 1.0 - s
    m = zlast
    for z in zs:
        m = jnp.maximum(m, z)
    se = jnp.exp(zlast - m)
    for z in zs:
        se = se + jnp.exp(z - m)
    lse = m + jnp.log(se)

    d = d.reshape(NB, 1)
    e = e.reshape(NB, 1)
    kio = jax.lax.broadcasted_iota(jnp.int32, (NB, K), 1)
    c = jnp.where(kio <= d, lse - zlast, 0.0)

    ze = zs[Q - 1]
    for q in range(Q - 2, -1, -1):
        ze = jnp.where(e == q + 1, zs[q], ze)
    corr = jnp.where((kio == d) & (e != 0), zlast - ze, 0.0)
    return jnp.sum(c) + jnp.sum(corr)


def _tc_body(*refs, Q, K):
    phi_refs = refs[:_S]
    d_refs = refs[_S:2 * _S]
    e_refs = refs[2 * _S:3 * _S]
    out_ref = refs[3 * _S]
    total = 0.0
    for s in range(_S):
        total += _partial_sum(phi_refs[s][...], d_refs[s][0, 0, :],
                              e_refs[s][0, 0, :], Q=Q, K=K)

    @pl.when(pl.program_id(0) == 0)
    def _init():
        out_ref[0, 0] = 0.0

    out_ref[0, 0] += total


def kernel(phi, idx_durations, events):
    N, Q, K = phi.shape
    NB = 1024
    nblk = N // NB
    g = nblk // _S
    phi2 = phi.reshape(N, Q * K)
    d3 = idx_durations.astype(jnp.int32).reshape(nblk, 1, NB)
    e3 = events.astype(jnp.int32).reshape(nblk, 1, NB)

    def phi_map(s):
        return lambda i: (i + s * g, 0)

    def de_map(s):
        return lambda i: (i + s * g, 0, 0)

    out = pl.pallas_call(
        functools.partial(_tc_body, Q=Q, K=K),
        grid=(g,),
        in_specs=(
            [pl.BlockSpec((NB, Q * K), phi_map(s)) for s in range(_S)]
            + [pl.BlockSpec((1, 1, NB), de_map(s)) for s in range(_S)]
            + [pl.BlockSpec((1, 1, NB), de_map(s)) for s in range(_S)]
        ),
        out_specs=pl.BlockSpec(memory_space=pltpu.SMEM),
        out_shape=jax.ShapeDtypeStruct((1, 1), jnp.float32),
        compiler_params=pltpu.CompilerParams(
            dimension_semantics=("arbitrary",),
        ),
    )(*([phi2] * _S + [d3] * _S + [e3] * _S))
    return out[0, 0] / N


# speedup vs baseline: 1.3657x; 1.0355x over previous
import functools

import jax
import jax.numpy as jnp
from jax.experimental import pallas as pl
from jax.experimental.pallas import tpu as pltpu

_S = 2


def _partial_sum(p, d, e, *, Q, K):
    NB = p.shape[0]
    zs = [p[:, q * K:(q + 1) * K] for q in range(Q)]
    s = zs[0]
    for q in range(1, Q):
        s = s + zs[q]
    zlast = 1.0 - s
    se = jnp.exp(zlast)
    for z in zs:
        se = se + jnp.exp(z)
    lse = jnp.log(se)

    d = d.reshape(NB, 1)
    e = e.reshape(NB, 1)
    kio = jax.lax.broadcasted_iota(jnp.int32, (NB, K), 1)
    c = jnp.where(kio <= d, lse - zlast, 0.0)

    ze = zs[Q - 1]
    for q in range(Q - 2, -1, -1):
        ze = jnp.where(e == q + 1, zs[q], ze)
    corr = jnp.where((kio == d) & (e != 0), zlast - ze, 0.0)
    return jnp.sum(c) + jnp.sum(corr)


def _tc_body(*refs, Q, K):
    phi_refs = refs[:_S]
    d_refs = refs[_S:2 * _S]
    e_refs = refs[2 * _S:3 * _S]
    out_ref = refs[3 * _S]
    total = 0.0
    for s in range(_S):
        total += _partial_sum(phi_refs[s][...], d_refs[s][0, 0, :],
                              e_refs[s][0, 0, :], Q=Q, K=K)

    @pl.when(pl.program_id(0) == 0)
    def _init():
        out_ref[0, 0] = 0.0

    out_ref[0, 0] += total


def kernel(phi, idx_durations, events):
    N, Q, K = phi.shape
    NB = 1024
    nblk = N // NB
    g = nblk // _S
    phi2 = phi.reshape(N, Q * K)
    d3 = idx_durations.astype(jnp.int32).reshape(nblk, 1, NB)
    e3 = events.astype(jnp.int32).reshape(nblk, 1, NB)

    def phi_map(s):
        return lambda i: (i + s * g, 0)

    def de_map(s):
        return lambda i: (i + s * g, 0, 0)

    out = pl.pallas_call(
        functools.partial(_tc_body, Q=Q, K=K),
        grid=(g,),
        in_specs=(
            [pl.BlockSpec((NB, Q * K), phi_map(s)) for s in range(_S)]
            + [pl.BlockSpec((1, 1, NB), de_map(s)) for s in range(_S)]
            + [pl.BlockSpec((1, 1, NB), de_map(s)) for s in range(_S)]
        ),
        out_specs=pl.BlockSpec(memory_space=pltpu.SMEM),
        out_shape=jax.ShapeDtypeStruct((1, 1), jnp.float32),
        compiler_params=pltpu.CompilerParams(
            dimension_semantics=("arbitrary",),
        ),
    )(*([phi2] * _S + [d3] * _S + [e3] * _S))
    return out[0, 0] / N


# speedup vs baseline: 1.3774x; 1.0086x over previous
import functools

import jax
import jax.numpy as jnp
from jax.experimental import pallas as pl
from jax.experimental.pallas import tpu as pltpu

_S = 2


def _partial_sum(p, d, e, *, Q, K):
    NB = p.shape[0]
    zs = [p[:, q * K:(q + 1) * K] for q in range(Q)]
    s = zs[0]
    for q in range(1, Q):
        s = s + zs[q]
    zlast = 1.0 - s
    se = jnp.exp(zlast)
    for z in zs:
        se = se + jnp.exp(z)
    lse = jnp.log(se)

    d = d.reshape(NB, 1)
    e = e.reshape(NB, 1)
    kio = jax.lax.broadcasted_iota(jnp.int32, (NB, K), 1)
    ze = zs[Q - 1]
    for q in range(Q - 2, -1, -1):
        ze = jnp.where(e == q + 1, zs[q], ze)
    sub = jnp.where((kio == d) & (e != 0), ze, zlast)
    c = jnp.where(kio <= d, lse - sub, 0.0)
    return jnp.sum(c)


def _tc_body(*refs, Q, K):
    phi_refs = refs[:_S]
    d_refs = refs[_S:2 * _S]
    e_refs = refs[2 * _S:3 * _S]
    out_ref = refs[3 * _S]
    total = 0.0
    for s in range(_S):
        total += _partial_sum(phi_refs[s][...], d_refs[s][0, 0, :],
                              e_refs[s][0, 0, :], Q=Q, K=K)

    @pl.when(pl.program_id(0) == 0)
    def _init():
        out_ref[0, 0] = 0.0

    out_ref[0, 0] += total


def kernel(phi, idx_durations, events):
    N, Q, K = phi.shape
    NB = 1024
    nblk = N // NB
    g = nblk // _S
    phi2 = phi.reshape(N, Q * K)
    d3 = idx_durations.astype(jnp.int32).reshape(nblk, 1, NB)
    e3 = events.astype(jnp.int32).reshape(nblk, 1, NB)

    def phi_map(s):
        return lambda i: (i + s * g, 0)

    def de_map(s):
        return lambda i: (i + s * g, 0, 0)

    out = pl.pallas_call(
        functools.partial(_tc_body, Q=Q, K=K),
        grid=(g,),
        in_specs=(
            [pl.BlockSpec((NB, Q * K), phi_map(s)) for s in range(_S)]
            + [pl.BlockSpec((1, 1, NB), de_map(s)) for s in range(_S)]
            + [pl.BlockSpec((1, 1, NB), de_map(s)) for s in range(_S)]
        ),
        out_specs=pl.BlockSpec(memory_space=pltpu.SMEM),
        out_shape=jax.ShapeDtypeStruct((1, 1), jnp.float32),
        compiler_params=pltpu.CompilerParams(
            dimension_semantics=("arbitrary",),
        ),
    )(*([phi2] * _S + [d3] * _S + [e3] * _S))
    return out[0, 0] / N
